# baseline (speedup 1.0000x reference)
import functools

import jax
import jax.numpy as jnp
import numpy as np
from jax import lax
from jax.experimental import pallas as pl
from jax.experimental.pallas import tpu as pltpu
from jax.experimental.pallas import tpu_sc as plsc

N, E, C = 10000, 320000, 128
NC, NS = 2, 16
N_PAD = 10240
CHUNK = 128

PT1 = 10240
EP1 = PT1 * NS
PT2 = 20480
EP2 = PT2 * NS

_ma = np.zeros((C, C), np.float32)
for _c in range(C):
    for _k in range(_c - 12, _c + 13):
        _ma[min(max(_k, 0), C - 1), _c] += 1.0 / 25.0
_MA = _ma


def _pad_edges(src, dst, per_core, n_cores_split):
    e_half = E // n_cores_split
    srcs, dsts = [], []
    for c in range(n_cores_split):
        s = src[c * e_half:(c + 1) * e_half]
        d = dst[c * e_half:(c + 1) * e_half]
        npad = per_core - e_half
        j = jnp.arange(npad, dtype=jnp.int32)
        srcs.append(jnp.concatenate([s, j % 997]))
        dsts.append(jnp.concatenate([d, N + (j % (N_PAD - N))]))
    return jnp.stack(srcs), jnp.stack(dsts)


def _make_sc_agg(t_rows, pt, with_deg):
    n_chunks = pt // CHUNK
    rows_per_tile = N_PAD // NS
    mesh = plsc.VectorSubcoreMesh(core_axis_name="c", subcore_axis_name="s")
    out_type = [jax.ShapeDtypeStruct((NC, N_PAD, C), jnp.float32)]
    if with_deg:
        out_type.append(jax.ShapeDtypeStruct((NC, N_PAD), jnp.float32))
    scratch = [
        pltpu.VMEM_SHARED((N_PAD, C), jnp.float32),
        pltpu.VMEM((CHUNK,), jnp.int32),
        pltpu.VMEM((CHUNK,), jnp.int32),
        pltpu.VMEM((CHUNK, C), jnp.float32),
        pltpu.SemaphoreType.DMA,
    ]
    if with_deg:
        scratch.insert(1, pltpu.VMEM_SHARED((N_PAD,), jnp.float32))
        scratch.append(pltpu.VMEM((CHUNK,), jnp.float32))

    @functools.partial(pl.kernel, out_type=out_type, mesh=mesh,
                       scratch_types=scratch, name="sc_edge_agg")
    def k(table_h, srcs_h, dsts_h, zeros_h, *refs):
        if with_deg:
            (zeros1_h, acc_out, deg_out, acc_sh, deg_sh, sidx_v, didx_v,
             rows_v, sem, ones_v) = refs
        else:
            acc_out, acc_sh, sidx_v, didx_v, rows_v, sem = refs
        c = lax.axis_index("c")
        s = lax.axis_index("s")

        zslc = pl.ds(s * rows_per_tile, rows_per_tile)
        pltpu.sync_copy(zeros_h.at[zslc], acc_sh.at[zslc])
        if with_deg:
            pltpu.sync_copy(zeros1_h.at[zslc], deg_sh.at[zslc])
            for i in range(CHUNK // 16):
                ones_v[pl.ds(i * 16, 16)] = jnp.full((16,), 1.0, jnp.float32)
        plsc.subcore_barrier()

        base = s * pt

        def body(j, _):
            off = base + j * CHUNK
            pltpu.sync_copy(srcs_h.at[c, pl.ds(off, CHUNK)], sidx_v)
            pltpu.sync_copy(dsts_h.at[c, pl.ds(off, CHUNK)], didx_v)
            pltpu.async_copy(table_h.at[sidx_v], rows_v, sem).wait()
            pltpu.sync_copy(rows_v, acc_sh.at[didx_v], add=True)
            if with_deg:
                pltpu.sync_copy(ones_v, deg_sh.at[didx_v], add=True)
            return ()

        lax.fori_loop(0, n_chunks, body, ())
        plsc.subcore_barrier()

        pltpu.sync_copy(acc_sh.at[zslc], acc_out.at[c, zslc])
        if with_deg:
            pltpu.sync_copy(deg_sh.at[zslc], deg_out.at[c, zslc])

    return k


_sc_agg1 = _make_sc_agg(N, PT1, with_deg=True)
_sc_agg2 = _make_sc_agg(2 * N, PT2, with_deg=False)


_HI = lax.Precision.HIGHEST
BR = 2000
NB = N // BR

def _row_spec(shape):
    return pl.BlockSpec(shape, lambda i: (i,) + (0,) * (len(shape) - 1))


def _stk_spec(shape):
    return pl.BlockSpec(shape, lambda i: (0, i, 0))


def _fix_spec(shape):
    return pl.BlockSpec(shape, lambda i: (0,) * len(shape))


def _stats_accum(stats_ref, i, a, b):
    @pl.when(i == 0)
    def _():
        stats_ref[...] = jnp.zeros(stats_ref.shape, stats_ref.dtype)
    stats_ref[0:1] += jnp.sum(a, axis=0, keepdims=True)
    stats_ref[1:2] += jnp.sum(a * a, axis=0, keepdims=True)
    stats_ref[2:3] += jnp.sum(b, axis=0, keepdims=True)
    stats_ref[3:4] += jnp.sum(b * b, axis=0, keepdims=True)


def _bn_coefs(stats_ref, row, g, bt):
    mu = stats_ref[row:row + 1] * (1.0 / N)
    var = stats_ref[row + 1:row + 2] * (1.0 / N) - mu * mu
    scale = g * lax.rsqrt(var + 1e-5)
    return scale, bt - mu * scale


def _tc_pre1(x_ref, a0_ref, a1_ref, d0_ref, d1_ref, ma_ref,
             w1l, w1r, b1, w2l, w2r, b2,
             hp_ref, st_ref, r_ref, stats_ref):
    i = pl.program_id(0)
    x = x_ref[...]
    ma = ma_ref[...]
    t = jnp.dot(x, ma, precision=_HI)
    s = x - t
    aggx = a0_ref[...] + a1_ref[...]
    r = 1.0 / jnp.maximum(d0_ref[...] + d1_ref[...], 1.0)
    r_ref[...] = r
    aggt = jnp.dot(aggx, ma, precision=_HI)
    h1p = jnp.dot((aggx - aggt) * r, w1l[...], precision=_HI) + b1[...] \
        + jnp.dot(s, w1r[...], precision=_HI)
    h2p = jnp.dot(aggt * r, w2l[...], precision=_HI) + b2[...] \
        + jnp.dot(t, w2r[...], precision=_HI)
    hp_ref[0] = h1p
    hp_ref[1] = h2p
    st_ref[0] = s
    st_ref[1] = t
    _stats_accum(stats_ref, i, h1p, h2p)


def _tc_bnrelu(hp_ref, stats_ref, g1, bt1, g2, bt2, h_ref):
    sc1, sh1 = _bn_coefs(stats_ref, 0, g1[...], bt1[...])
    sc2, sh2 = _bn_coefs(stats_ref, 2, g2[...], bt2[...])
    h_ref[0] = jax.nn.relu(hp_ref[0] * sc1 + sh1)
    h_ref[1] = jax.nn.relu(hp_ref[1] * sc2 + sh2)


def _tc_pre2(h_ref, a0_ref, a1_ref, r_ref, w1l, w1r, b1, w2l, w2r, b2,
             op_ref, stats_ref):
    i = pl.program_id(0)
    r = r_ref[...]
    o1p = jnp.dot(a0_ref[...] * r, w1l[...], precision=_HI) + b1[...] \
        + jnp.dot(h_ref[0], w1r[...], precision=_HI)
    o2p = jnp.dot(a1_ref[...] * r, w2l[...], precision=_HI) + b2[...] \
        + jnp.dot(h_ref[1], w2r[...], precision=_HI)
    op_ref[0] = o1p
    op_ref[1] = o2p
    _stats_accum(stats_ref, i, o1p, o2p)


def _tc_final(op_ref, st_ref, stats_ref, g1, bt1, g2, bt2, out_ref):
    sc1, sh1 = _bn_coefs(stats_ref, 0, g1[...], bt1[...])
    sc2, sh2 = _bn_coefs(stats_ref, 2, g2[...], bt2[...])
    o1 = jax.nn.relu(op_ref[0] * sc1 + sh1 + st_ref[0])
    o2 = jax.nn.relu(op_ref[1] * sc2 + sh2 + st_ref[1])
    out_ref[...] = o1 + o2


def kernel(x, edge_index, W11l, W11r, b11, g11, bt11, W12l, W12r, b12, g12,
           bt12, W21l, W21r, b21, g21, bt21, W22l, W22r, b22, g22, bt22):
    src = edge_index[0]
    dst = edge_index[1]
    srcs1, dsts1 = _pad_edges(src, dst, EP1, 2)
    src_p, dst_p = _pad_edges(src, dst, EP2, 1)
    srcs2 = jnp.concatenate([src_p, src_p + N])
    dsts2 = jnp.concatenate([dst_p, dst_p])
    zeros = jnp.zeros((N_PAD, C), jnp.float32)
    zeros1 = jnp.zeros((N_PAD,), jnp.float32)

    mat = _fix_spec((C, C))
    vec = _fix_spec((1, C))
    stats_spec = _fix_spec((8, C))
    row = _row_spec((BR, C))
    row1 = _row_spec((BR, 1))
    stk = _stk_spec((2, BR, C))

    aggx, degp = _sc_agg1(x, srcs1, dsts1, zeros, zeros1)

    hp, st, r, stats1 = pl.pallas_call(
        _tc_pre1,
        grid=(NB,),
        in_specs=[row, row, row, row1, row1, mat,
                  mat, mat, vec, mat, mat, vec],
        out_specs=[stk, stk, row1, stats_spec],
        out_shape=[jax.ShapeDtypeStruct((2, N, C), jnp.float32),
                   jax.ShapeDtypeStruct((2, N, C), jnp.float32),
                   jax.ShapeDtypeStruct((N, 1), jnp.float32),
                   jax.ShapeDtypeStruct((8, C), jnp.float32)],
        name="tc_pre1",
    )(x, aggx[0, :N], aggx[1, :N], degp[0, :N, None], degp[1, :N, None], _MA,
      W11l.T, W11r.T, b11[None, :], W21l.T, W21r.T, b21[None, :])

    h = pl.pallas_call(
        _tc_bnrelu,
        grid=(NB,),
        in_specs=[stk, stats_spec, vec, vec, vec, vec],
        out_specs=stk,
        out_shape=jax.ShapeDtypeStruct((2, N, C), jnp.float32),
        name="tc_bnrelu",
    )(hp, stats1, g11[None, :], bt11[None, :], g21[None, :], bt21[None, :])

    (agg2,) = _sc_agg2(h.reshape(2 * N, C), srcs2, dsts2, zeros)

    op, stats2 = pl.pallas_call(
        _tc_pre2,
        grid=(NB,),
        in_specs=[stk, row, row, row1, mat, mat, vec, mat, mat, vec],
        out_specs=[stk, stats_spec],
        out_shape=[jax.ShapeDtypeStruct((2, N, C), jnp.float32),
                   jax.ShapeDtypeStruct((8, C), jnp.float32)],
        name="tc_pre2",
    )(h, agg2[0, :N], agg2[1, :N], r,
      W12l.T, W12r.T, b12[None, :], W22l.T, W22r.T, b22[None, :])

    return pl.pallas_call(
        _tc_final,
        grid=(NB,),
        in_specs=[stk, stk, stats_spec, vec, vec, vec, vec],
        out_specs=row,
        out_shape=jax.ShapeDtypeStruct((N, C), jnp.float32),
        name="tc_final",
    )(op, st, stats2, g12[None, :], bt12[None, :], g22[None, :], bt22[None, :])


# speedup vs baseline: 1.7503x; 1.7503x over previous
import functools

import jax
import jax.numpy as jnp
import numpy as np
from jax import lax
from jax.experimental import pallas as pl
from jax.experimental.pallas import tpu as pltpu
from jax.experimental.pallas import tpu_sc as plsc

N, E, C = 10000, 320000, 128
NC, NS = 2, 16
N_PAD = 10240
CHUNK = 128

PT1 = 10240
EP1 = PT1 * NS
PT2 = 20480
EP2 = PT2 * NS

_ma = np.zeros((C, C), np.float32)
for _c in range(C):
    for _k in range(_c - 12, _c + 13):
        _ma[min(max(_k, 0), C - 1), _c] += 1.0 / 25.0
_MA = _ma


def _pad_edges(src, dst, per_core, n_cores_split):
    e_half = E // n_cores_split
    srcs, dsts = [], []
    for c in range(n_cores_split):
        s = src[c * e_half:(c + 1) * e_half]
        d = dst[c * e_half:(c + 1) * e_half]
        npad = per_core - e_half
        j = jnp.arange(npad, dtype=jnp.int32)
        srcs.append(jnp.concatenate([s, j % 997]))
        dsts.append(jnp.concatenate([d, N + (j % (N_PAD - N))]))
    return jnp.stack(srcs), jnp.stack(dsts)


def _make_sc_agg(t_rows, pt, with_deg):
    n_chunks = pt // CHUNK
    rows_per_tile = N_PAD // NS
    mesh = plsc.VectorSubcoreMesh(core_axis_name="c", subcore_axis_name="s")
    out_type = [jax.ShapeDtypeStruct((NC, N_PAD, C), jnp.float32)]
    if with_deg:
        out_type.append(jax.ShapeDtypeStruct((NC, N_PAD), jnp.float32))
    scratch = [
        pltpu.VMEM_SHARED((N_PAD, C), jnp.float32),
        pltpu.VMEM((2, CHUNK), jnp.int32),
        pltpu.VMEM((2, CHUNK), jnp.int32),
        pltpu.VMEM((CHUNK, C), jnp.float32),
        pltpu.VMEM((CHUNK, C), jnp.float32),
        pltpu.SemaphoreType.DMA,
        pltpu.SemaphoreType.DMA,
        pltpu.SemaphoreType.DMA,
        pltpu.SemaphoreType.DMA,
    ]
    if with_deg:
        scratch.insert(1, pltpu.VMEM_SHARED((N_PAD,), jnp.float32))
        scratch.append(pltpu.VMEM((CHUNK,), jnp.float32))

    @functools.partial(pl.kernel, out_type=out_type, mesh=mesh,
                       scratch_types=scratch, name="sc_edge_agg")
    def k(table_h, sd_h, zeros_h, *refs):
        if with_deg:
            (zeros1_h, acc_out, deg_out, acc_sh, deg_sh, ib0, ib1,
             rows0_v, rows1_v, semA, semB, sem0, sem1, ones_v) = refs
        else:
            (acc_out, acc_sh, ib0, ib1, rows0_v, rows1_v,
             semA, semB, sem0, sem1) = refs
        c = lax.axis_index("c")
        s = lax.axis_index("s")

        zslc = pl.ds(s * rows_per_tile, rows_per_tile)
        pltpu.sync_copy(zeros_h.at[zslc], acc_sh.at[zslc])
        if with_deg:
            pltpu.sync_copy(zeros1_h.at[zslc], deg_sh.at[zslc])
            for i in range(CHUNK // 16):
                ones_v[pl.ds(i * 16, 16)] = jnp.full((16,), 1.0, jnp.float32)
        plsc.subcore_barrier()

        def fire_idx(j, ib, sem):
            pltpu.async_copy(sd_h.at[c, s, j], ib, sem)

        def drain_idx(ib, sem):
            pltpu.make_async_copy(sd_h.at[c, s, 0], ib, sem).wait()

        def fire_rows(ib, rows_v, sem):
            pltpu.async_copy(table_h.at[ib.at[0]], rows_v, sem)

        def drain_rows(rows_v, sem):
            pltpu.make_async_copy(table_h.at[pl.ds(0, CHUNK)], rows_v,
                                  sem).wait()

        def scat(rows_v, ib):
            pltpu.sync_copy(rows_v, acc_sh.at[ib.at[1]], add=True)
            if with_deg:
                pltpu.sync_copy(ones_v, deg_sh.at[ib.at[1]], add=True)

        fire_idx(0, ib0, semA)
        fire_idx(1, ib1, semB)
        drain_idx(ib0, semA)
        fire_rows(ib0, rows0_v, sem0)

        def body(j2, _):
            j = 2 * j2
            drain_idx(ib1, semB)
            fire_rows(ib1, rows1_v, sem1)
            drain_rows(rows0_v, sem0)
            scat(rows0_v, ib0)
            fire_idx(jnp.minimum(j + 2, n_chunks - 1), ib0, semA)
            drain_idx(ib0, semA)
            fire_rows(ib0, rows0_v, sem0)
            drain_rows(rows1_v, sem1)
            scat(rows1_v, ib1)
            fire_idx(jnp.minimum(j + 3, n_chunks - 1), ib1, semB)
            return ()

        lax.fori_loop(0, n_chunks // 2, body, ())
        drain_idx(ib1, semB)
        drain_rows(rows0_v, sem0)
        plsc.subcore_barrier()

        pltpu.sync_copy(acc_sh.at[zslc], acc_out.at[c, zslc])
        if with_deg:
            pltpu.sync_copy(deg_sh.at[zslc], deg_out.at[c, zslc])

    return k


_sc_agg1 = _make_sc_agg(N, PT1, with_deg=True)
_sc_agg2 = _make_sc_agg(2 * N, PT2, with_deg=False)


_HI = lax.Precision.HIGHEST
BR = 2000
NB = N // BR

def _row_spec(shape):
    return pl.BlockSpec(shape, lambda i: (i,) + (0,) * (len(shape) - 1))


def _stk_spec(shape):
    return pl.BlockSpec(shape, lambda i: (0, i, 0))


def _fix_spec(shape):
    return pl.BlockSpec(shape, lambda i: (0,) * len(shape))


def _stats_accum(stats_ref, i, a, b):
    @pl.when(i == 0)
    def _():
        stats_ref[...] = jnp.zeros(stats_ref.shape, stats_ref.dtype)
    stats_ref[0:1] += jnp.sum(a, axis=0, keepdims=True)
    stats_ref[1:2] += jnp.sum(a * a, axis=0, keepdims=True)
    stats_ref[2:3] += jnp.sum(b, axis=0, keepdims=True)
    stats_ref[3:4] += jnp.sum(b * b, axis=0, keepdims=True)


def _bn_coefs(stats_ref, row, g, bt):
    mu = stats_ref[row:row + 1] * (1.0 / N)
    var = stats_ref[row + 1:row + 2] * (1.0 / N) - mu * mu
    scale = g * lax.rsqrt(var + 1e-5)
    return scale, bt - mu * scale


def _tc_pre1(x_ref, a0_ref, a1_ref, d0_ref, d1_ref, ma_ref,
             w1l, w1r, b1, w2l, w2r, b2,
             hp_ref, st_ref, r_ref, stats_ref):
    i = pl.program_id(0)
    x = x_ref[...]
    ma = ma_ref[...]
    t = jnp.dot(x, ma, precision=_HI)
    s = x - t
    aggx = a0_ref[...] + a1_ref[...]
    r = 1.0 / jnp.maximum(d0_ref[...] + d1_ref[...], 1.0)
    r_ref[...] = r
    aggt = jnp.dot(aggx, ma, precision=_HI)
    h1p = jnp.dot((aggx - aggt) * r, w1l[...], precision=_HI) + b1[...] \
        + jnp.dot(s, w1r[...], precision=_HI)
    h2p = jnp.dot(aggt * r, w2l[...], precision=_HI) + b2[...] \
        + jnp.dot(t, w2r[...], precision=_HI)
    hp_ref[0] = h1p
    hp_ref[1] = h2p
    st_ref[0] = s
    st_ref[1] = t
    _stats_accum(stats_ref, i, h1p, h2p)


def _tc_bnrelu(hp_ref, stats_ref, g1, bt1, g2, bt2, h_ref):
    sc1, sh1 = _bn_coefs(stats_ref, 0, g1[...], bt1[...])
    sc2, sh2 = _bn_coefs(stats_ref, 2, g2[...], bt2[...])
    h_ref[0] = jax.nn.relu(hp_ref[0] * sc1 + sh1)
    h_ref[1] = jax.nn.relu(hp_ref[1] * sc2 + sh2)


def _tc_pre2(h_ref, a0_ref, a1_ref, r_ref, w1l, w1r, b1, w2l, w2r, b2,
             op_ref, stats_ref):
    i = pl.program_id(0)
    r = r_ref[...]
    o1p = jnp.dot(a0_ref[...] * r, w1l[...], precision=_HI) + b1[...] \
        + jnp.dot(h_ref[0], w1r[...], precision=_HI)
    o2p = jnp.dot(a1_ref[...] * r, w2l[...], precision=_HI) + b2[...] \
        + jnp.dot(h_ref[1], w2r[...], precision=_HI)
    op_ref[0] = o1p
    op_ref[1] = o2p
    _stats_accum(stats_ref, i, o1p, o2p)


def _tc_final(op_ref, st_ref, stats_ref, g1, bt1, g2, bt2, out_ref):
    sc1, sh1 = _bn_coefs(stats_ref, 0, g1[...], bt1[...])
    sc2, sh2 = _bn_coefs(stats_ref, 2, g2[...], bt2[...])
    o1 = jax.nn.relu(op_ref[0] * sc1 + sh1 + st_ref[0])
    o2 = jax.nn.relu(op_ref[1] * sc2 + sh2 + st_ref[1])
    out_ref[...] = o1 + o2


def kernel(x, edge_index, W11l, W11r, b11, g11, bt11, W12l, W12r, b12, g12,
           bt12, W21l, W21r, b21, g21, bt21, W22l, W22r, b22, g22, bt22):
    src = edge_index[0]
    dst = edge_index[1]
    srcs1, dsts1 = _pad_edges(src, dst, EP1, 2)
    sd1 = jnp.stack([srcs1.reshape(NC, NS, PT1 // CHUNK, CHUNK),
                     dsts1.reshape(NC, NS, PT1 // CHUNK, CHUNK)], axis=3)
    src_p, dst_p = _pad_edges(src, dst, EP2, 1)
    srcs2 = jnp.concatenate([src_p, src_p + N])
    dsts2 = jnp.concatenate([dst_p, dst_p])
    sd2 = jnp.stack([srcs2.reshape(NC, NS, PT2 // CHUNK, CHUNK),
                     dsts2.reshape(NC, NS, PT2 // CHUNK, CHUNK)], axis=3)
    zeros = jnp.zeros((N_PAD, C), jnp.float32)
    zeros1 = jnp.zeros((N_PAD,), jnp.float32)

    mat = _fix_spec((C, C))
    vec = _fix_spec((1, C))
    stats_spec = _fix_spec((8, C))
    row = _row_spec((BR, C))
    row1 = _row_spec((BR, 1))
    stk = _stk_spec((2, BR, C))

    aggx, degp = _sc_agg1(x, sd1, zeros, zeros1)

    hp, st, r, stats1 = pl.pallas_call(
        _tc_pre1,
        grid=(NB,),
        in_specs=[row, row, row, row1, row1, mat,
                  mat, mat, vec, mat, mat, vec],
        out_specs=[stk, stk, row1, stats_spec],
        out_shape=[jax.ShapeDtypeStruct((2, N, C), jnp.float32),
                   jax.ShapeDtypeStruct((2, N, C), jnp.float32),
                   jax.ShapeDtypeStruct((N, 1), jnp.float32),
                   jax.ShapeDtypeStruct((8, C), jnp.float32)],
        name="tc_pre1",
    )(x, aggx[0, :N], aggx[1, :N], degp[0, :N, None], degp[1, :N, None], _MA,
      W11l.T, W11r.T, b11[None, :], W21l.T, W21r.T, b21[None, :])

    h = pl.pallas_call(
        _tc_bnrelu,
        grid=(NB,),
        in_specs=[stk, stats_spec, vec, vec, vec, vec],
        out_specs=stk,
        out_shape=jax.ShapeDtypeStruct((2, N, C), jnp.float32),
        name="tc_bnrelu",
    )(hp, stats1, g11[None, :], bt11[None, :], g21[None, :], bt21[None, :])

    (agg2,) = _sc_agg2(h.reshape(2 * N, C), sd2, zeros)

    op, stats2 = pl.pallas_call(
        _tc_pre2,
        grid=(NB,),
        in_specs=[stk, row, row, row1, mat, mat, vec, mat, mat, vec],
        out_specs=[stk, stats_spec],
        out_shape=[jax.ShapeDtypeStruct((2, N, C), jnp.float32),
                   jax.ShapeDtypeStruct((8, C), jnp.float32)],
        name="tc_pre2",
    )(h, agg2[0, :N], agg2[1, :N], r,
      W12l.T, W12r.T, b12[None, :], W22l.T, W22r.T, b22[None, :])

    return pl.pallas_call(
        _tc_final,
        grid=(NB,),
        in_specs=[stk, stk, stats_spec, vec, vec, vec, vec],
        out_specs=row,
        out_shape=jax.ShapeDtypeStruct((N, C), jnp.float32),
        name="tc_final",
    )(op, st, stats2, g12[None, :], bt12[None, :], g22[None, :], bt22[None, :])


# speedup vs baseline: 1.9279x; 1.1014x over previous
import functools

import jax
import jax.numpy as jnp
import numpy as np
from jax import lax
from jax.experimental import pallas as pl
from jax.experimental.pallas import tpu as pltpu
from jax.experimental.pallas import tpu_sc as plsc

N, E, C = 10000, 320000, 128
NC, NS = 2, 16
N_PAD = 10240
CHUNK = 128

K_IDX = 8

PT1 = 10240
EP1 = PT1 * NS
PT2 = 20480
EP2 = PT2 * NS

_ma = np.zeros((C, C), np.float32)
for _c in range(C):
    for _k in range(_c - 12, _c + 13):
        _ma[min(max(_k, 0), C - 1), _c] += 1.0 / 25.0
_MA = _ma


def _pad_edges(src, dst, per_core, n_cores_split):
    e_half = E // n_cores_split
    srcs, dsts = [], []
    for c in range(n_cores_split):
        s = src[c * e_half:(c + 1) * e_half]
        d = dst[c * e_half:(c + 1) * e_half]
        npad = per_core - e_half
        j = jnp.arange(npad, dtype=jnp.int32)
        srcs.append(jnp.concatenate([s, j % 997]))
        dsts.append(jnp.concatenate([d, N + (j % (N_PAD - N))]))
    return jnp.stack(srcs), jnp.stack(dsts)


def _make_sc_agg(t_rows, pt, with_deg):
    n_chunks = pt // CHUNK
    n_batches = n_chunks // K_IDX
    rows_per_tile = N_PAD // NS
    mesh = plsc.VectorSubcoreMesh(core_axis_name="c", subcore_axis_name="s")
    out_type = [jax.ShapeDtypeStruct((NC, N_PAD, C), jnp.float32)]
    if with_deg:
        out_type.append(jax.ShapeDtypeStruct((NC, N_PAD), jnp.float32))
    scratch = [
        pltpu.VMEM_SHARED((N_PAD, C), jnp.float32),
        pltpu.VMEM((K_IDX, 2, CHUNK), jnp.int32),
        pltpu.VMEM((K_IDX, 2, CHUNK), jnp.int32),
        pltpu.VMEM((CHUNK, C), jnp.float32),
        pltpu.VMEM((CHUNK, C), jnp.float32),
        pltpu.SemaphoreType.DMA,
        pltpu.SemaphoreType.DMA,
        pltpu.SemaphoreType.DMA,
        pltpu.SemaphoreType.DMA,
    ]
    if with_deg:
        scratch.insert(1, pltpu.VMEM_SHARED((N_PAD,), jnp.float32))
        scratch.append(pltpu.VMEM((CHUNK,), jnp.float32))

    @functools.partial(pl.kernel, out_type=out_type, mesh=mesh,
                       scratch_types=scratch, name="sc_edge_agg")
    def k(table_h, sd_h, zeros_h, *refs):
        if with_deg:
            (zeros1_h, acc_out, deg_out, acc_sh, deg_sh, ibA, ibB,
             rows0_v, rows1_v, semA, semB, sem0, sem1, ones_v) = refs
        else:
            (acc_out, acc_sh, ibA, ibB, rows0_v, rows1_v,
             semA, semB, sem0, sem1) = refs
        c = lax.axis_index("c")
        s = lax.axis_index("s")

        zslc = pl.ds(s * rows_per_tile, rows_per_tile)
        pltpu.sync_copy(zeros_h.at[zslc], acc_sh.at[zslc])
        if with_deg:
            pltpu.sync_copy(zeros1_h.at[zslc], deg_sh.at[zslc])
            for i in range(CHUNK // 16):
                ones_v[pl.ds(i * 16, 16)] = jnp.full((16,), 1.0, jnp.float32)
        plsc.subcore_barrier()

        rows = (rows0_v, rows1_v)
        rsems = (sem0, sem1)

        def fire_b(g, ib, sem):
            pltpu.async_copy(sd_h.at[c, s, g], ib, sem)

        def drain_b(ib, sem):
            pltpu.make_async_copy(sd_h.at[c, s, 0], ib, sem).wait()

        def fire_rows(idx_ref, rows_v, sem):
            pltpu.async_copy(table_h.at[idx_ref], rows_v, sem)

        def drain_rows(rows_v, sem):
            pltpu.make_async_copy(table_h.at[pl.ds(0, CHUNK)], rows_v,
                                  sem).wait()

        def scat(rows_v, ib, kk):
            pltpu.sync_copy(rows_v, acc_sh.at[ib.at[kk, 1]], add=True)
            if with_deg:
                pltpu.sync_copy(ones_v, deg_sh.at[ib.at[kk, 1]], add=True)

        fire_b(0, ibA, semA)
        fire_b(1, ibB, semB)
        drain_b(ibA, semA)
        fire_rows(ibA.at[0, 0], rows0_v, sem0)

        def half(ib_cur, sem_cur, ib_nxt, sem_nxt, reload_g):
            for kk in range(K_IDX):
                cur, csem = rows[kk % 2], rsems[kk % 2]
                nxt, nsem = rows[(kk + 1) % 2], rsems[(kk + 1) % 2]
                if kk < K_IDX - 1:
                    fire_rows(ib_cur.at[kk + 1, 0], nxt, nsem)
                else:
                    drain_b(ib_nxt, sem_nxt)
                    fire_rows(ib_nxt.at[0, 0], nxt, nsem)
                drain_rows(cur, csem)
                scat(cur, ib_cur, kk)
            fire_b(reload_g, ib_cur, sem_cur)

        def body(g2, _):
            g = 2 * g2
            half(ibA, semA, ibB, semB, jnp.minimum(g + 2, n_batches - 1))
            half(ibB, semB, ibA, semA, jnp.minimum(g + 3, n_batches - 1))
            return ()

        lax.fori_loop(0, n_batches // 2, body, ())
        drain_b(ibB, semB)
        drain_rows(rows0_v, sem0)
        plsc.subcore_barrier()

        pltpu.sync_copy(acc_sh.at[zslc], acc_out.at[c, zslc])
        if with_deg:
            pltpu.sync_copy(deg_sh.at[zslc], deg_out.at[c, zslc])

    return k


_sc_agg1 = _make_sc_agg(N, PT1, with_deg=True)
_sc_agg2 = _make_sc_agg(2 * N, PT2, with_deg=False)


_HI = lax.Precision.HIGHEST
BR = 2000
NB = N // BR

def _row_spec(shape):
    return pl.BlockSpec(shape, lambda i: (i,) + (0,) * (len(shape) - 1))


def _stk_spec(shape):
    return pl.BlockSpec(shape, lambda i: (0, i, 0))


def _fix_spec(shape):
    return pl.BlockSpec(shape, lambda i: (0,) * len(shape))


def _stats_accum(stats_ref, i, a, b):
    @pl.when(i == 0)
    def _():
        stats_ref[...] = jnp.zeros(stats_ref.shape, stats_ref.dtype)
    stats_ref[0:1] += jnp.sum(a, axis=0, keepdims=True)
    stats_ref[1:2] += jnp.sum(a * a, axis=0, keepdims=True)
    stats_ref[2:3] += jnp.sum(b, axis=0, keepdims=True)
    stats_ref[3:4] += jnp.sum(b * b, axis=0, keepdims=True)


def _bn_coefs(stats_ref, row, g, bt):
    mu = stats_ref[row:row + 1] * (1.0 / N)
    var = stats_ref[row + 1:row + 2] * (1.0 / N) - mu * mu
    scale = g * lax.rsqrt(var + 1e-5)
    return scale, bt - mu * scale


def _tc_pre1(x_ref, a0_ref, a1_ref, d0_ref, d1_ref, ma_ref,
             w1l, w1r, b1, w2l, w2r, b2,
             hp_ref, st_ref, r_ref, stats_ref):
    i = pl.program_id(0)
    x = x_ref[...]
    ma = ma_ref[...]
    t = jnp.dot(x, ma, precision=_HI)
    s = x - t
    aggx = a0_ref[...] + a1_ref[...]
    r = 1.0 / jnp.maximum(d0_ref[...] + d1_ref[...], 1.0)
    r_ref[...] = r
    aggt = jnp.dot(aggx, ma, precision=_HI)
    h1p = jnp.dot((aggx - aggt) * r, w1l[...], precision=_HI) + b1[...] \
        + jnp.dot(s, w1r[...], precision=_HI)
    h2p = jnp.dot(aggt * r, w2l[...], precision=_HI) + b2[...] \
        + jnp.dot(t, w2r[...], precision=_HI)
    hp_ref[0] = h1p
    hp_ref[1] = h2p
    st_ref[0] = s
    st_ref[1] = t
    _stats_accum(stats_ref, i, h1p, h2p)


def _tc_bnrelu(hp_ref, stats_ref, g1, bt1, g2, bt2, h_ref):
    sc1, sh1 = _bn_coefs(stats_ref, 0, g1[...], bt1[...])
    sc2, sh2 = _bn_coefs(stats_ref, 2, g2[...], bt2[...])
    h_ref[0] = jax.nn.relu(hp_ref[0] * sc1 + sh1)
    h_ref[1] = jax.nn.relu(hp_ref[1] * sc2 + sh2)


def _tc_pre2(h_ref, a0_ref, a1_ref, r_ref, w1l, w1r, b1, w2l, w2r, b2,
             op_ref, stats_ref):
    i = pl.program_id(0)
    r = r_ref[...]
    o1p = jnp.dot(a0_ref[...] * r, w1l[...], precision=_HI) + b1[...] \
        + jnp.dot(h_ref[0], w1r[...], precision=_HI)
    o2p = jnp.dot(a1_ref[...] * r, w2l[...], precision=_HI) + b2[...] \
        + jnp.dot(h_ref[1], w2r[...], precision=_HI)
    op_ref[0] = o1p
    op_ref[1] = o2p
    _stats_accum(stats_ref, i, o1p, o2p)


def _tc_final(op_ref, st_ref, stats_ref, g1, bt1, g2, bt2, out_ref):
    sc1, sh1 = _bn_coefs(stats_ref, 0, g1[...], bt1[...])
    sc2, sh2 = _bn_coefs(stats_ref, 2, g2[...], bt2[...])
    o1 = jax.nn.relu(op_ref[0] * sc1 + sh1 + st_ref[0])
    o2 = jax.nn.relu(op_ref[1] * sc2 + sh2 + st_ref[1])
    out_ref[...] = o1 + o2


def kernel(x, edge_index, W11l, W11r, b11, g11, bt11, W12l, W12r, b12, g12,
           bt12, W21l, W21r, b21, g21, bt21, W22l, W22r, b22, g22, bt22):
    src = edge_index[0]
    dst = edge_index[1]
    srcs1, dsts1 = _pad_edges(src, dst, EP1, 2)
    sd1 = jnp.stack([srcs1.reshape(NC, NS, PT1 // CHUNK, CHUNK),
                     dsts1.reshape(NC, NS, PT1 // CHUNK, CHUNK)],
                    axis=3).reshape(NC, NS, -1, K_IDX, 2, CHUNK)
    src_p, dst_p = _pad_edges(src, dst, EP2, 1)
    srcs2 = jnp.concatenate([src_p, src_p + N])
    dsts2 = jnp.concatenate([dst_p, dst_p])
    sd2 = jnp.stack([srcs2.reshape(NC, NS, PT2 // CHUNK, CHUNK),
                     dsts2.reshape(NC, NS, PT2 // CHUNK, CHUNK)],
                    axis=3).reshape(NC, NS, -1, K_IDX, 2, CHUNK)
    zeros = jnp.zeros((N_PAD, C), jnp.float32)
    zeros1 = jnp.zeros((N_PAD,), jnp.float32)

    mat = _fix_spec((C, C))
    vec = _fix_spec((1, C))
    stats_spec = _fix_spec((8, C))
    row = _row_spec((BR, C))
    row1 = _row_spec((BR, 1))
    stk = _stk_spec((2, BR, C))

    aggx, degp = _sc_agg1(x, sd1, zeros, zeros1)

    hp, st, r, stats1 = pl.pallas_call(
        _tc_pre1,
        grid=(NB,),
        in_specs=[row, row, row, row1, row1, mat,
                  mat, mat, vec, mat, mat, vec],
        out_specs=[stk, stk, row1, stats_spec],
        out_shape=[jax.ShapeDtypeStruct((2, N, C), jnp.float32),
                   jax.ShapeDtypeStruct((2, N, C), jnp.float32),
                   jax.ShapeDtypeStruct((N, 1), jnp.float32),
                   jax.ShapeDtypeStruct((8, C), jnp.float32)],
        name="tc_pre1",
    )(x, aggx[0, :N], aggx[1, :N], degp[0, :N, None], degp[1, :N, None], _MA,
      W11l.T, W11r.T, b11[None, :], W21l.T, W21r.T, b21[None, :])

    h = pl.pallas_call(
        _tc_bnrelu,
        grid=(NB,),
        in_specs=[stk, stats_spec, vec, vec, vec, vec],
        out_specs=stk,
        out_shape=jax.ShapeDtypeStruct((2, N, C), jnp.float32),
        name="tc_bnrelu",
    )(hp, stats1, g11[None, :], bt11[None, :], g21[None, :], bt21[None, :])

    (agg2,) = _sc_agg2(h.reshape(2 * N, C), sd2, zeros)

    op, stats2 = pl.pallas_call(
        _tc_pre2,
        grid=(NB,),
        in_specs=[stk, row, row, row1, mat, mat, vec, mat, mat, vec],
        out_specs=[stk, stats_spec],
        out_shape=[jax.ShapeDtypeStruct((2, N, C), jnp.float32),
                   jax.ShapeDtypeStruct((8, C), jnp.float32)],
        name="tc_pre2",
    )(h, agg2[0, :N], agg2[1, :N], r,
      W12l.T, W12r.T, b12[None, :], W22l.T, W22r.T, b22[None, :])

    return pl.pallas_call(
        _tc_final,
        grid=(NB,),
        in_specs=[stk, stk, stats_spec, vec, vec, vec, vec],
        out_specs=row,
        out_shape=jax.ShapeDtypeStruct((N, C), jnp.float32),
        name="tc_final",
    )(op, st, stats2, g12[None, :], bt12[None, :], g22[None, :], bt22[None, :])


# speedup vs baseline: 2.0403x; 1.0583x over previous
import functools

import jax
import jax.numpy as jnp
import numpy as np
from jax import lax
from jax.experimental import pallas as pl
from jax.experimental.pallas import tpu as pltpu
from jax.experimental.pallas import tpu_sc as plsc

N, E, C = 10000, 320000, 128
NC, NS = 2, 16
N_PAD = 10240
CHUNK = 128

K_IDX = 8

PT1 = 10240
EP1 = PT1 * NS
PT2 = 20480
EP2 = PT2 * NS

_ma = np.zeros((C, C), np.float32)
for _c in range(C):
    for _k in range(_c - 12, _c + 13):
        _ma[min(max(_k, 0), C - 1), _c] += 1.0 / 25.0
_MA = _ma


def _pad_edges(src, dst, per_core, n_cores_split):
    e_half = E // n_cores_split
    srcs, dsts = [], []
    for c in range(n_cores_split):
        s = src[c * e_half:(c + 1) * e_half]
        d = dst[c * e_half:(c + 1) * e_half]
        npad = per_core - e_half
        j = jnp.arange(npad, dtype=jnp.int32)
        srcs.append(jnp.concatenate([s, j % 997]))
        dsts.append(jnp.concatenate([d, N + (j % (N_PAD - N))]))
    return jnp.stack(srcs), jnp.stack(dsts)


def _make_sc_agg(t_rows, pt, with_deg):
    n_chunks = pt // CHUNK
    n_batches = n_chunks // K_IDX
    rows_per_tile = N_PAD // NS
    mesh = plsc.VectorSubcoreMesh(core_axis_name="c", subcore_axis_name="s")
    out_type = [jax.ShapeDtypeStruct((NC, N_PAD, C), jnp.float32)]
    if with_deg:
        out_type.append(jax.ShapeDtypeStruct((NC, N_PAD), jnp.float32))
    scratch = [
        pltpu.VMEM_SHARED((N_PAD, C), jnp.float32),
        pltpu.VMEM((K_IDX, 2, CHUNK), jnp.int32),
        pltpu.VMEM((K_IDX, 2, CHUNK), jnp.int32),
        pltpu.VMEM((CHUNK, C), jnp.float32),
        pltpu.VMEM((CHUNK, C), jnp.float32),
        pltpu.SemaphoreType.DMA,
        pltpu.SemaphoreType.DMA,
        pltpu.SemaphoreType.DMA,
        pltpu.SemaphoreType.DMA,
    ]
    if with_deg:
        scratch.insert(1, pltpu.VMEM_SHARED((N_PAD,), jnp.float32))
        scratch.append(pltpu.VMEM((CHUNK,), jnp.float32))

    @functools.partial(pl.kernel, out_type=out_type, mesh=mesh,
                       scratch_types=scratch, name="sc_edge_agg")
    def k(table_h, sd_h, zeros_h, *refs):
        if with_deg:
            (zeros1_h, acc_out, deg_out, acc_sh, deg_sh, ibA, ibB,
             rows0_v, rows1_v, semA, semB, sem0, sem1, ones_v) = refs
        else:
            (acc_out, acc_sh, ibA, ibB, rows0_v, rows1_v,
             semA, semB, sem0, sem1) = refs
        c = lax.axis_index("c")
        s = lax.axis_index("s")

        zslc = pl.ds(s * rows_per_tile, rows_per_tile)
        pltpu.sync_copy(zeros_h.at[zslc], acc_sh.at[zslc])
        if with_deg:
            pltpu.sync_copy(zeros1_h.at[zslc], deg_sh.at[zslc])
            for i in range(CHUNK // 16):
                ones_v[pl.ds(i * 16, 16)] = jnp.full((16,), 1.0, jnp.float32)
        plsc.subcore_barrier()

        rows = (rows0_v, rows1_v)
        rsems = (sem0, sem1)

        def fire_b(g, ib, sem):
            pltpu.async_copy(sd_h.at[c, s, g], ib, sem)

        def drain_b(ib, sem):
            pltpu.make_async_copy(sd_h.at[c, s, 0], ib, sem).wait()

        def fire_rows(idx_ref, rows_v, sem):
            pltpu.async_copy(table_h.at[idx_ref], rows_v, sem)

        def drain_rows(rows_v, sem):
            pltpu.make_async_copy(table_h.at[pl.ds(0, CHUNK)], rows_v,
                                  sem).wait()

        def scat(rows_v, ib, kk):
            pltpu.sync_copy(rows_v, acc_sh.at[ib.at[kk, 1]], add=True)
            if with_deg:
                pltpu.sync_copy(ones_v, deg_sh.at[ib.at[kk, 1]], add=True)

        fire_b(0, ibA, semA)
        fire_b(1, ibB, semB)
        drain_b(ibA, semA)
        fire_rows(ibA.at[0, 0], rows0_v, sem0)

        def half(ib_cur, sem_cur, ib_nxt, sem_nxt, reload_g):
            for kk in range(K_IDX):
                cur, csem = rows[kk % 2], rsems[kk % 2]
                nxt, nsem = rows[(kk + 1) % 2], rsems[(kk + 1) % 2]
                if kk < K_IDX - 1:
                    fire_rows(ib_cur.at[kk + 1, 0], nxt, nsem)
                else:
                    drain_b(ib_nxt, sem_nxt)
                    fire_rows(ib_nxt.at[0, 0], nxt, nsem)
                drain_rows(cur, csem)
                scat(cur, ib_cur, kk)
            fire_b(reload_g, ib_cur, sem_cur)

        def body(g2, _):
            g = 2 * g2
            half(ibA, semA, ibB, semB, jnp.minimum(g + 2, n_batches - 1))
            half(ibB, semB, ibA, semA, jnp.minimum(g + 3, n_batches - 1))
            return ()

        lax.fori_loop(0, n_batches // 2, body, ())
        drain_b(ibB, semB)
        drain_rows(rows0_v, sem0)
        plsc.subcore_barrier()

        pltpu.sync_copy(acc_sh.at[zslc], acc_out.at[c, zslc])
        if with_deg:
            pltpu.sync_copy(deg_sh.at[zslc], deg_out.at[c, zslc])

    return k


_sc_agg1 = _make_sc_agg(N, PT1, with_deg=True)
_sc_agg2 = _make_sc_agg(2 * N, PT2, with_deg=False)


_HI = lax.Precision.HIGHEST
BR = 2000
NB = N // BR


def _stats_accum(stats_ref, i, a, b):
    @pl.when(i == 0)
    def _():
        stats_ref[...] = jnp.zeros(stats_ref.shape, stats_ref.dtype)
    stats_ref[0:1] += jnp.sum(a, axis=0, keepdims=True)
    stats_ref[1:2] += jnp.sum(a * a, axis=0, keepdims=True)
    stats_ref[2:3] += jnp.sum(b, axis=0, keepdims=True)
    stats_ref[3:4] += jnp.sum(b * b, axis=0, keepdims=True)


def _bn_coefs(stats_ref, row, g, bt):
    mu = stats_ref[row:row + 1] * (1.0 / N)
    var = stats_ref[row + 1:row + 2] * (1.0 / N) - mu * mu
    scale = g * lax.rsqrt(var + 1e-5)
    return scale, bt - mu * scale


def _tc_a(x_ref, a0_ref, a1_ref, d0_ref, d1_ref, ma_ref,
          w1l, w1r, b1, w2l, w2r, b2, g1, bt1, g2, bt2,
          h_ref, hp_scr, stats_scr):
    p = pl.program_id(0)
    i = pl.program_id(1)

    @pl.when(p == 0)
    def _():
        x = x_ref[...]
        ma = ma_ref[...]
        t = jnp.dot(x, ma, precision=_HI)
        s = x - t
        aggx = a0_ref[0] + a1_ref[0]
        r = 1.0 / jnp.maximum(d0_ref[0] + d1_ref[0], 1.0)
        aggt = jnp.dot(aggx, ma, precision=_HI)
        h1p = jnp.dot((aggx - aggt) * r, w1l[...], precision=_HI) + b1[...] \
            + jnp.dot(s, w1r[...], precision=_HI)
        h2p = jnp.dot(aggt * r, w2l[...], precision=_HI) + b2[...] \
            + jnp.dot(t, w2r[...], precision=_HI)
        hp_scr[0, pl.ds(i * BR, BR)] = h1p
        hp_scr[1, pl.ds(i * BR, BR)] = h2p
        _stats_accum(stats_scr, i, h1p, h2p)

    @pl.when(p == 1)
    def _():
        sc1, sh1 = _bn_coefs(stats_scr, 0, g1[...], bt1[...])
        sc2, sh2 = _bn_coefs(stats_scr, 2, g2[...], bt2[...])
        h_ref[0] = jax.nn.relu(hp_scr[0, pl.ds(i * BR, BR)] * sc1 + sh1)
        h_ref[1] = jax.nn.relu(hp_scr[1, pl.ds(i * BR, BR)] * sc2 + sh2)


def _tc_b(h_ref, a0_ref, a1_ref, d0_ref, d1_ref, x_ref, ma_ref,
          w1l, w1r, b1, w2l, w2r, b2, g1, bt1, g2, bt2,
          out_ref, op_scr, stats_scr):
    p = pl.program_id(0)
    i = pl.program_id(1)

    @pl.when(p == 0)
    def _():
        r = 1.0 / jnp.maximum(d0_ref[0] + d1_ref[0], 1.0)
        o1p = jnp.dot(a0_ref[0] * r, w1l[...], precision=_HI) + b1[...] \
            + jnp.dot(h_ref[0], w1r[...], precision=_HI)
        o2p = jnp.dot(a1_ref[0] * r, w2l[...], precision=_HI) + b2[...] \
            + jnp.dot(h_ref[1], w2r[...], precision=_HI)
        op_scr[0, pl.ds(i * BR, BR)] = o1p
        op_scr[1, pl.ds(i * BR, BR)] = o2p
        _stats_accum(stats_scr, i, o1p, o2p)

    @pl.when(p == 1)
    def _():
        x = x_ref[...]
        t = jnp.dot(x, ma_ref[...], precision=_HI)
        s = x - t
        sc1, sh1 = _bn_coefs(stats_scr, 0, g1[...], bt1[...])
        sc2, sh2 = _bn_coefs(stats_scr, 2, g2[...], bt2[...])
        o1 = jax.nn.relu(op_scr[0, pl.ds(i * BR, BR)] * sc1 + sh1 + s)
        o2 = jax.nn.relu(op_scr[1, pl.ds(i * BR, BR)] * sc2 + sh2 + t)
        out_ref[...] = o1 + o2


def kernel(x, edge_index, W11l, W11r, b11, g11, bt11, W12l, W12r, b12, g12,
           bt12, W21l, W21r, b21, g21, bt21, W22l, W22r, b22, g22, bt22):
    src = edge_index[0]
    dst = edge_index[1]
    srcs1, dsts1 = _pad_edges(src, dst, EP1, 2)
    sd1 = jnp.stack([srcs1.reshape(NC, NS, PT1 // CHUNK, CHUNK),
                     dsts1.reshape(NC, NS, PT1 // CHUNK, CHUNK)],
                    axis=3).reshape(NC, NS, -1, K_IDX, 2, CHUNK)
    src_p, dst_p = _pad_edges(src, dst, EP2, 1)
    srcs2 = jnp.concatenate([src_p, src_p + N])
    dsts2 = jnp.concatenate([dst_p, dst_p])
    sd2 = jnp.stack([srcs2.reshape(NC, NS, PT2 // CHUNK, CHUNK),
                     dsts2.reshape(NC, NS, PT2 // CHUNK, CHUNK)],
                    axis=3).reshape(NC, NS, -1, K_IDX, 2, CHUNK)
    zeros = jnp.zeros((N_PAD, C), jnp.float32)
    zeros1 = jnp.zeros((N_PAD,), jnp.float32)

    mat = pl.BlockSpec((C, C), lambda p, i: (0, 0))
    vec = pl.BlockSpec((1, C), lambda p, i: (0, 0))
    row_p0 = pl.BlockSpec((BR, C), lambda p, i: (i * (1 - p), 0))
    row_p1 = pl.BlockSpec((BR, C), lambda p, i: (i * p, 0))
    stk_p0 = pl.BlockSpec((2, BR, C), lambda p, i: (0, i * (1 - p), 0))
    stk_p1 = pl.BlockSpec((2, BR, C), lambda p, i: (0, i * p, 0))

    def core(c, shape):
        return pl.BlockSpec(shape, lambda p, i, c=c: (c, i * (1 - p), 0))

    scratch = [pltpu.VMEM((2, N, C), jnp.float32),
               pltpu.VMEM((8, C), jnp.float32)]

    aggx, degp = _sc_agg1(x, sd1, zeros, zeros1)
    degp3 = degp[..., None]

    h = pl.pallas_call(
        _tc_a,
        grid=(2, NB),
        in_specs=[row_p0, core(0, (1, BR, C)), core(1, (1, BR, C)),
                  core(0, (1, BR, 1)), core(1, (1, BR, 1)), mat,
                  mat, mat, vec, mat, mat, vec, vec, vec, vec, vec],
        out_specs=stk_p1,
        out_shape=jax.ShapeDtypeStruct((2, N, C), jnp.float32),
        scratch_shapes=scratch,
        name="tc_a",
    )(x, aggx, aggx, degp3, degp3, _MA,
      W11l.T, W11r.T, b11[None, :], W21l.T, W21r.T, b21[None, :],
      g11[None, :], bt11[None, :], g21[None, :], bt21[None, :])

    (agg2,) = _sc_agg2(h.reshape(2 * N, C), sd2, zeros)

    return pl.pallas_call(
        _tc_b,
        grid=(2, NB),
        in_specs=[stk_p0, core(0, (1, BR, C)), core(1, (1, BR, C)),
                  core(0, (1, BR, 1)), core(1, (1, BR, 1)), row_p1, mat,
                  mat, mat, vec, mat, mat, vec, vec, vec, vec, vec],
        out_specs=row_p1,
        out_shape=jax.ShapeDtypeStruct((N, C), jnp.float32),
        scratch_shapes=scratch,
        name="tc_b",
    )(h, agg2, agg2, degp3, degp3, x, _MA,
      W12l.T, W12r.T, b12[None, :], W22l.T, W22r.T, b22[None, :],
      g12[None, :], bt12[None, :], g22[None, :], bt22[None, :])


# speedup vs baseline: 2.0405x; 1.0001x over previous
import functools

import jax
import jax.numpy as jnp
import numpy as np
from jax import lax
from jax.experimental import pallas as pl
from jax.experimental.pallas import tpu as pltpu
from jax.experimental.pallas import tpu_sc as plsc

N, E, C = 10000, 320000, 128
NC, NS = 2, 16
N_PAD = 10240
CHUNK = 128

K_IDX = 8

PT1 = 10240
EP1 = PT1 * NS
PT2 = 20480
EP2 = PT2 * NS

_ma = np.zeros((C, C), np.float32)
for _c in range(C):
    for _k in range(_c - 12, _c + 13):
        _ma[min(max(_k, 0), C - 1), _c] += 1.0 / 25.0
_MA = _ma


def _pad_edges(src, dst, per_core, n_cores_split):
    e_half = E // n_cores_split
    srcs, dsts = [], []
    for c in range(n_cores_split):
        s = src[c * e_half:(c + 1) * e_half]
        d = dst[c * e_half:(c + 1) * e_half]
        npad = per_core - e_half
        j = jnp.arange(npad, dtype=jnp.int32)
        srcs.append(jnp.concatenate([s, j % 997]))
        dsts.append(jnp.concatenate([d, N + (j % (N_PAD - N))]))
    return jnp.stack(srcs), jnp.stack(dsts)


def _make_sc_agg(pt, with_deg, inner=(C,), dtype=jnp.float32):
    n_chunks = pt // CHUNK
    n_batches = n_chunks // K_IDX
    rows_per_tile = N_PAD // NS
    mesh = plsc.VectorSubcoreMesh(core_axis_name="c", subcore_axis_name="s")
    out_type = [jax.ShapeDtypeStruct((NC, N_PAD) + inner, dtype)]
    if with_deg:
        out_type.append(jax.ShapeDtypeStruct((NC, N_PAD), jnp.float32))
    scratch = [
        pltpu.VMEM_SHARED((N_PAD,) + inner, dtype),
        pltpu.VMEM((K_IDX, 2, CHUNK), jnp.int32),
        pltpu.VMEM((K_IDX, 2, CHUNK), jnp.int32),
        pltpu.VMEM((CHUNK,) + inner, dtype),
        pltpu.VMEM((CHUNK,) + inner, dtype),
        pltpu.SemaphoreType.DMA,
        pltpu.SemaphoreType.DMA,
        pltpu.SemaphoreType.DMA,
        pltpu.SemaphoreType.DMA,
        pltpu.SemaphoreType.DMA,
        pltpu.SemaphoreType.DMA,
    ]
    if with_deg:
        scratch.insert(1, pltpu.VMEM_SHARED((N_PAD,), jnp.float32))
        scratch.append(pltpu.VMEM((CHUNK,), jnp.float32))

    @functools.partial(pl.kernel, out_type=out_type, mesh=mesh,
                       scratch_types=scratch, name="sc_edge_agg")
    def k(table_h, sd_h, zeros_h, *refs):
        if with_deg:
            (zeros1_h, acc_out, deg_out, acc_sh, deg_sh, ibA, ibB,
             r0, r1, semA, semB, g0, g1, s0, s1, ones_v) = refs
        else:
            (acc_out, acc_sh, ibA, ibB, r0, r1,
             semA, semB, g0, g1, s0, s1) = refs
        c = lax.axis_index("c")
        s = lax.axis_index("s")

        zslc = pl.ds(s * rows_per_tile, rows_per_tile)
        pltpu.sync_copy(zeros_h.at[zslc], acc_sh.at[zslc])
        if with_deg:
            pltpu.sync_copy(zeros1_h.at[zslc], deg_sh.at[zslc])
            for i in range(CHUNK // 16):
                ones_v[pl.ds(i * 16, 16)] = jnp.full((16,), 1.0, jnp.float32)
        plsc.subcore_barrier()

        rows = (r0, r1)
        rsems = (g0, g1)
        ssems = (s0, s1)

        def fire_b(g, ib, sem):
            pltpu.async_copy(sd_h.at[c, s, g], ib, sem)

        def drain_b(ib, sem):
            pltpu.make_async_copy(sd_h.at[c, s, 0], ib, sem).wait()

        def fire_rows(idx_ref, rows_v, sem):
            pltpu.async_copy(table_h.at[idx_ref], rows_v, sem)

        def drain_rows(rows_v, sem):
            pltpu.make_async_copy(table_h.at[pl.ds(0, CHUNK)], rows_v,
                                  sem).wait()

        def scat_fire(b, ib, kk):
            pltpu.sync_copy(rows[b], acc_sh.at[ib.at[kk, 1]], add=True)
            if with_deg:
                pltpu.sync_copy(ones_v, deg_sh.at[ib.at[kk, 1]], add=True)

        def drain_scat(b):
            pass

        fire_b(0, ibA, semA)
        drain_b(ibA, semA)
        fire_rows(ibA.at[0, 0], rows[0], rsems[0])

        def half(ib_cur, ib_nxt, sem_nxt, reload_g, first):
            for kk in range(K_IDX):
                b = kk % 2
                nb = (kk + 1) % 2
                if kk == 3:
                    fire_b(reload_g, ib_nxt, sem_nxt)
                if not (first and kk < 1):
                    drain_scat(nb)
                if kk < K_IDX - 1:
                    fire_rows(ib_cur.at[kk + 1, 0], rows[nb], rsems[nb])
                else:
                    drain_b(ib_nxt, sem_nxt)
                    fire_rows(ib_nxt.at[0, 0], rows[nb], rsems[nb])
                drain_rows(rows[b], rsems[b])
                scat_fire(b, ib_cur, kk)

        half(ibA, ibB, semB, 1, True)
        half(ibB, ibA, semA, jnp.minimum(2, n_batches - 1), False)

        def body(g2, _):
            g = 2 * g2
            half(ibA, ibB, semB, g + 1, False)
            half(ibB, ibA, semA, jnp.minimum(g + 2, n_batches - 1), False)
            return ()

        lax.fori_loop(1, n_batches // 2, body, ())
        drain_rows(rows[0], rsems[0])
        drain_scat(1)
        plsc.subcore_barrier()

        pltpu.sync_copy(acc_sh.at[zslc], acc_out.at[c, zslc])
        if with_deg:
            pltpu.sync_copy(deg_sh.at[zslc], deg_out.at[c, zslc])

    return k


_sc_agg1 = _make_sc_agg(PT1, with_deg=True)
_sc_agg2 = _make_sc_agg(PT2, with_deg=False)


_HI = lax.Precision.HIGHEST
BR = 2000
NB = N // BR


def _stats_accum(stats_ref, i, a, b):
    @pl.when(i == 0)
    def _():
        stats_ref[...] = jnp.zeros(stats_ref.shape, stats_ref.dtype)
    stats_ref[0:1] += jnp.sum(a, axis=0, keepdims=True)
    stats_ref[1:2] += jnp.sum(a * a, axis=0, keepdims=True)
    stats_ref[2:3] += jnp.sum(b, axis=0, keepdims=True)
    stats_ref[3:4] += jnp.sum(b * b, axis=0, keepdims=True)


def _bn_coefs(stats_ref, row, g, bt):
    mu = stats_ref[row:row + 1] * (1.0 / N)
    var = stats_ref[row + 1:row + 2] * (1.0 / N) - mu * mu
    scale = g * lax.rsqrt(var + 1e-5)
    return scale, bt - mu * scale


def _tc_a(x_ref, a0_ref, a1_ref, d0_ref, d1_ref, ma_ref,
          w1l, w1r, b1, w2l, w2r, b2, g1, bt1, g2, bt2,
          h_ref, hp_scr, stats_scr):
    p = pl.program_id(0)
    i = pl.program_id(1)

    @pl.when(p == 0)
    def _():
        x = x_ref[...]
        ma = ma_ref[...]
        t = jnp.dot(x, ma, precision=_HI)
        s = x - t
        aggx = a0_ref[0] + a1_ref[0]
        r = 1.0 / jnp.maximum(d0_ref[0] + d1_ref[0], 1.0)
        aggt = jnp.dot(aggx, ma, precision=_HI)
        h1p = jnp.dot((aggx - aggt) * r, w1l[...], precision=_HI) + b1[...] \
            + jnp.dot(s, w1r[...], precision=_HI)
        h2p = jnp.dot(aggt * r, w2l[...], precision=_HI) + b2[...] \
            + jnp.dot(t, w2r[...], precision=_HI)
        hp_scr[0, pl.ds(i * BR, BR)] = h1p
        hp_scr[1, pl.ds(i * BR, BR)] = h2p
        _stats_accum(stats_scr, i, h1p, h2p)

    @pl.when(p == 1)
    def _():
        sc1, sh1 = _bn_coefs(stats_scr, 0, g1[...], bt1[...])
        sc2, sh2 = _bn_coefs(stats_scr, 2, g2[...], bt2[...])
        h_ref[0] = jax.nn.relu(hp_scr[0, pl.ds(i * BR, BR)] * sc1 + sh1)
        h_ref[1] = jax.nn.relu(hp_scr[1, pl.ds(i * BR, BR)] * sc2 + sh2)


def _tc_b(h_ref, a0_ref, a1_ref, d0_ref, d1_ref, x_ref, ma_ref,
          w1l, w1r, b1, w2l, w2r, b2, g1, bt1, g2, bt2,
          out_ref, op_scr, stats_scr):
    p = pl.program_id(0)
    i = pl.program_id(1)

    @pl.when(p == 0)
    def _():
        r = 1.0 / jnp.maximum(d0_ref[0] + d1_ref[0], 1.0)
        o1p = jnp.dot(a0_ref[0] * r, w1l[...], precision=_HI) + b1[...] \
            + jnp.dot(h_ref[0], w1r[...], precision=_HI)
        o2p = jnp.dot(a1_ref[0] * r, w2l[...], precision=_HI) + b2[...] \
            + jnp.dot(h_ref[1], w2r[...], precision=_HI)
        op_scr[0, pl.ds(i * BR, BR)] = o1p
        op_scr[1, pl.ds(i * BR, BR)] = o2p
        _stats_accum(stats_scr, i, o1p, o2p)

    @pl.when(p == 1)
    def _():
        x = x_ref[...]
        t = jnp.dot(x, ma_ref[...], precision=_HI)
        s = x - t
        sc1, sh1 = _bn_coefs(stats_scr, 0, g1[...], bt1[...])
        sc2, sh2 = _bn_coefs(stats_scr, 2, g2[...], bt2[...])
        o1 = jax.nn.relu(op_scr[0, pl.ds(i * BR, BR)] * sc1 + sh1 + s)
        o2 = jax.nn.relu(op_scr[1, pl.ds(i * BR, BR)] * sc2 + sh2 + t)
        out_ref[...] = o1 + o2


def kernel(x, edge_index, W11l, W11r, b11, g11, bt11, W12l, W12r, b12, g12,
           bt12, W21l, W21r, b21, g21, bt21, W22l, W22r, b22, g22, bt22):
    src = edge_index[0]
    dst = edge_index[1]
    srcs1, dsts1 = _pad_edges(src, dst, EP1, 2)
    sd1 = jnp.stack([srcs1.reshape(NC, NS, PT1 // CHUNK, CHUNK),
                     dsts1.reshape(NC, NS, PT1 // CHUNK, CHUNK)],
                    axis=3).reshape(NC, NS, -1, K_IDX, 2, CHUNK)
    src_p, dst_p = _pad_edges(src, dst, EP2, 1)
    srcs2 = jnp.concatenate([src_p, src_p + N])
    dsts2 = jnp.concatenate([dst_p, dst_p])
    sd2 = jnp.stack([srcs2.reshape(NC, NS, PT2 // CHUNK, CHUNK),
                     dsts2.reshape(NC, NS, PT2 // CHUNK, CHUNK)],
                    axis=3).reshape(NC, NS, -1, K_IDX, 2, CHUNK)
    zeros = jnp.zeros((N_PAD, C), jnp.float32)
    zeros1 = jnp.zeros((N_PAD,), jnp.float32)

    mat = pl.BlockSpec((C, C), lambda p, i: (0, 0))
    vec = pl.BlockSpec((1, C), lambda p, i: (0, 0))
    row_p0 = pl.BlockSpec((BR, C), lambda p, i: (i * (1 - p), 0))
    row_p1 = pl.BlockSpec((BR, C), lambda p, i: (i * p, 0))
    stk_p0 = pl.BlockSpec((2, BR, C), lambda p, i: (0, i * (1 - p), 0))
    stk_p1 = pl.BlockSpec((2, BR, C), lambda p, i: (0, i * p, 0))

    def core(c, shape):
        return pl.BlockSpec(shape, lambda p, i, c=c: (c, i * (1 - p), 0))

    scratch = [pltpu.VMEM((2, N, C), jnp.float32),
               pltpu.VMEM((8, C), jnp.float32)]

    aggx, degp = _sc_agg1(x, sd1, zeros, zeros1)
    degp3 = degp[..., None]

    h = pl.pallas_call(
        _tc_a,
        grid=(2, NB),
        in_specs=[row_p0, core(0, (1, BR, C)), core(1, (1, BR, C)),
                  core(0, (1, BR, 1)), core(1, (1, BR, 1)), mat,
                  mat, mat, vec, mat, mat, vec, vec, vec, vec, vec],
        out_specs=stk_p1,
        out_shape=jax.ShapeDtypeStruct((2, N, C), jnp.float32),
        scratch_shapes=scratch,
        name="tc_a",
    )(x, aggx, aggx, degp3, degp3, _MA,
      W11l.T, W11r.T, b11[None, :], W21l.T, W21r.T, b21[None, :],
      g11[None, :], bt11[None, :], g21[None, :], bt21[None, :])

    (agg2,) = _sc_agg2(h.reshape(2 * N, C), sd2, zeros)

    return pl.pallas_call(
        _tc_b,
        grid=(2, NB),
        in_specs=[stk_p0, core(0, (1, BR, C)), core(1, (1, BR, C)),
                  core(0, (1, BR, 1)), core(1, (1, BR, 1)), row_p1, mat,
                  mat, mat, vec, mat, mat, vec, vec, vec, vec, vec],
        out_specs=row_p1,
        out_shape=jax.ShapeDtypeStruct((N, C), jnp.float32),
        scratch_shapes=scratch,
        name="tc_b",
    )(h, agg2, agg2, degp3, degp3, x, _MA,
      W12l.T, W12r.T, b12[None, :], W22l.T, W22r.T, b22[None, :],
      g12[None, :], bt12[None, :], g22[None, :], bt22[None, :])


# speedup vs baseline: 2.1178x; 1.0379x over previous
import functools

import jax
import jax.numpy as jnp
import numpy as np
from jax import lax
from jax.experimental import pallas as pl
from jax.experimental.pallas import tpu as pltpu
from jax.experimental.pallas import tpu_sc as plsc

N, E, C = 10000, 320000, 128
NC, NS = 2, 16
N_PAD = 10240
CHUNK = 128

K_IDX = 8

PT1 = 10240
EP1 = PT1 * NS
PT2 = 20480
EP2 = PT2 * NS

_ma = np.zeros((C, C), np.float32)
for _c in range(C):
    for _k in range(_c - 12, _c + 13):
        _ma[min(max(_k, 0), C - 1), _c] += 1.0 / 25.0
_MA = _ma


def _pad_edges(src, dst, per_core, n_cores_split):
    e_half = E // n_cores_split
    srcs, dsts = [], []
    for c in range(n_cores_split):
        s = src[c * e_half:(c + 1) * e_half]
        d = dst[c * e_half:(c + 1) * e_half]
        npad = per_core - e_half
        j = jnp.arange(npad, dtype=jnp.int32)
        srcs.append(jnp.concatenate([s, j % 997]))
        dsts.append(jnp.concatenate([d, N + (j % (N_PAD - N))]))
    return jnp.stack(srcs), jnp.stack(dsts)


def _make_sc_agg(pt, with_deg, inner=(C,), dtype=jnp.float32):
    n_chunks = pt // CHUNK
    n_batches = n_chunks // K_IDX
    rows_per_tile = N_PAD // NS
    mesh = plsc.VectorSubcoreMesh(core_axis_name="c", subcore_axis_name="s")
    out_type = [jax.ShapeDtypeStruct((NC, N_PAD) + inner, dtype)]
    if with_deg:
        out_type.append(jax.ShapeDtypeStruct((NC, N_PAD), jnp.float32))
    scratch = [
        pltpu.VMEM_SHARED((N_PAD,) + inner, dtype),
        pltpu.VMEM((K_IDX, 2, CHUNK), jnp.int32),
        pltpu.VMEM((K_IDX, 2, CHUNK), jnp.int32),
        pltpu.VMEM((CHUNK,) + inner, dtype),
        pltpu.VMEM((CHUNK,) + inner, dtype),
        pltpu.SemaphoreType.DMA,
        pltpu.SemaphoreType.DMA,
        pltpu.SemaphoreType.DMA,
        pltpu.SemaphoreType.DMA,
        pltpu.SemaphoreType.DMA,
        pltpu.SemaphoreType.DMA,
    ]
    if with_deg:
        scratch.insert(1, pltpu.VMEM_SHARED((N_PAD,), jnp.float32))
        scratch.append(pltpu.VMEM((CHUNK,), jnp.float32))

    @functools.partial(pl.kernel, out_type=out_type, mesh=mesh,
                       scratch_types=scratch, name="sc_edge_agg")
    def k(table_h, sd_h, zeros_h, *refs):
        if with_deg:
            (zeros1_h, acc_out, deg_out, acc_sh, deg_sh, ibA, ibB,
             r0, r1, semA, semB, g0, g1, s0, s1, ones_v) = refs
        else:
            (acc_out, acc_sh, ibA, ibB, r0, r1,
             semA, semB, g0, g1, s0, s1) = refs
        c = lax.axis_index("c")
        s = lax.axis_index("s")

        zslc = pl.ds(s * rows_per_tile, rows_per_tile)
        pltpu.sync_copy(zeros_h.at[zslc], acc_sh.at[zslc])
        if with_deg:
            pltpu.sync_copy(zeros1_h.at[zslc], deg_sh.at[zslc])
            for i in range(CHUNK // 16):
                ones_v[pl.ds(i * 16, 16)] = jnp.full((16,), 1.0, jnp.float32)
        plsc.subcore_barrier()

        rows = (r0, r1)
        rsems = (g0, g1)
        ssems = (s0, s1)

        def fire_b(g, ib, sem):
            pltpu.async_copy(sd_h.at[c, s, g], ib, sem)

        def drain_b(ib, sem):
            pltpu.make_async_copy(sd_h.at[c, s, 0], ib, sem).wait()

        def fire_rows(idx_ref, rows_v, sem):
            pltpu.async_copy(table_h.at[idx_ref], rows_v, sem)

        def drain_rows(rows_v, sem):
            pltpu.make_async_copy(table_h.at[pl.ds(0, CHUNK)], rows_v,
                                  sem).wait()

        def scat_fire(b, ib, kk):
            pltpu.sync_copy(rows[b], acc_sh.at[ib.at[kk, 1]], add=True)
            if with_deg:
                pltpu.sync_copy(ones_v, deg_sh.at[ib.at[kk, 1]], add=True)

        def drain_scat(b):
            pass

        fire_b(0, ibA, semA)
        drain_b(ibA, semA)
        fire_rows(ibA.at[0, 0], rows[0], rsems[0])

        def half(ib_cur, ib_nxt, sem_nxt, reload_g, first):
            for kk in range(K_IDX):
                b = kk % 2
                nb = (kk + 1) % 2
                if kk == 3:
                    fire_b(reload_g, ib_nxt, sem_nxt)
                if not (first and kk < 1):
                    drain_scat(nb)
                if kk < K_IDX - 1:
                    fire_rows(ib_cur.at[kk + 1, 0], rows[nb], rsems[nb])
                else:
                    drain_b(ib_nxt, sem_nxt)
                    fire_rows(ib_nxt.at[0, 0], rows[nb], rsems[nb])
                drain_rows(rows[b], rsems[b])
                scat_fire(b, ib_cur, kk)

        half(ibA, ibB, semB, 1, True)
        half(ibB, ibA, semA, jnp.minimum(2, n_batches - 1), False)

        def body(g2, _):
            g = 2 * g2
            half(ibA, ibB, semB, g + 1, False)
            half(ibB, ibA, semA, jnp.minimum(g + 2, n_batches - 1), False)
            return ()

        lax.fori_loop(1, n_batches // 2, body, ())
        drain_rows(rows[0], rsems[0])
        drain_scat(1)
        plsc.subcore_barrier()

        pltpu.sync_copy(acc_sh.at[zslc], acc_out.at[c, zslc])
        if with_deg:
            pltpu.sync_copy(deg_sh.at[zslc], deg_out.at[c, zslc])

    return k


_sc_agg1 = _make_sc_agg(PT1, with_deg=True)
_sc_agg2 = _make_sc_agg(PT2, with_deg=False)


_HI = lax.Precision.HIGHEST
BR = 2000
NB = N // BR


def _stats_accum(stats_ref, i, a, b):
    @pl.when(i == 0)
    def _():
        stats_ref[...] = jnp.zeros(stats_ref.shape, stats_ref.dtype)
    stats_ref[0:1] += jnp.sum(a, axis=0, keepdims=True)
    stats_ref[1:2] += jnp.sum(a * a, axis=0, keepdims=True)
    stats_ref[2:3] += jnp.sum(b, axis=0, keepdims=True)
    stats_ref[3:4] += jnp.sum(b * b, axis=0, keepdims=True)


def _bn_coefs(stats_ref, row, g, bt):
    mu = stats_ref[row:row + 1] * (1.0 / N)
    var = stats_ref[row + 1:row + 2] * (1.0 / N) - mu * mu
    scale = g * lax.rsqrt(var + 1e-5)
    return scale, bt - mu * scale


def _tc_r1(x_ref, ma_ref, w1r, w2r, b1, b2, out_ref):
    x = x_ref[...]
    t = jnp.dot(x, ma_ref[...], precision=_HI)
    s = x - t
    out_ref[0] = jnp.dot(s, w1r[...], precision=_HI) + b1[...]
    out_ref[1] = jnp.dot(t, w2r[...], precision=_HI) + b2[...]


def _tc_a(a0_ref, a1_ref, d0_ref, d1_ref, rhs_ref, ma_ref,
          w1l, w2l, g1, bt1, g2, bt2, h_ref, hp_scr, stats_scr):
    p = pl.program_id(0)
    i = pl.program_id(1)

    @pl.when(p == 0)
    def _():
        aggx = a0_ref[0] + a1_ref[0]
        r = 1.0 / jnp.maximum(d0_ref[0] + d1_ref[0], 1.0)
        aggt = jnp.dot(aggx, ma_ref[...], precision=_HI)
        h1p = jnp.dot((aggx - aggt) * r, w1l[...], precision=_HI) \
            + rhs_ref[0]
        h2p = jnp.dot(aggt * r, w2l[...], precision=_HI) + rhs_ref[1]
        hp_scr[0, pl.ds(i * BR, BR)] = h1p
        hp_scr[1, pl.ds(i * BR, BR)] = h2p
        _stats_accum(stats_scr, i, h1p, h2p)

    @pl.when(p == 1)
    def _():
        sc1, sh1 = _bn_coefs(stats_scr, 0, g1[...], bt1[...])
        sc2, sh2 = _bn_coefs(stats_scr, 2, g2[...], bt2[...])
        h_ref[0] = jax.nn.relu(hp_scr[0, pl.ds(i * BR, BR)] * sc1 + sh1)
        h_ref[1] = jax.nn.relu(hp_scr[1, pl.ds(i * BR, BR)] * sc2 + sh2)


def _tc_r2(h_ref, w1r, w2r, b1, b2, out_ref):
    out_ref[0] = jnp.dot(h_ref[0], w1r[...], precision=_HI) + b1[...]
    out_ref[1] = jnp.dot(h_ref[1], w2r[...], precision=_HI) + b2[...]


def _tc_b(a0_ref, a1_ref, d0_ref, d1_ref, rhs_ref, x_ref, ma_ref,
          w1l, w2l, g1, bt1, g2, bt2, out_ref, op_scr, stats_scr):
    p = pl.program_id(0)
    i = pl.program_id(1)

    @pl.when(p == 0)
    def _():
        r = 1.0 / jnp.maximum(d0_ref[0] + d1_ref[0], 1.0)
        o1p = jnp.dot(a0_ref[0] * r, w1l[...], precision=_HI) + rhs_ref[0]
        o2p = jnp.dot(a1_ref[0] * r, w2l[...], precision=_HI) + rhs_ref[1]
        op_scr[0, pl.ds(i * BR, BR)] = o1p
        op_scr[1, pl.ds(i * BR, BR)] = o2p
        _stats_accum(stats_scr, i, o1p, o2p)

    @pl.when(p == 1)
    def _():
        x = x_ref[...]
        t = jnp.dot(x, ma_ref[...], precision=_HI)
        s = x - t
        sc1, sh1 = _bn_coefs(stats_scr, 0, g1[...], bt1[...])
        sc2, sh2 = _bn_coefs(stats_scr, 2, g2[...], bt2[...])
        o1 = jax.nn.relu(op_scr[0, pl.ds(i * BR, BR)] * sc1 + sh1 + s)
        o2 = jax.nn.relu(op_scr[1, pl.ds(i * BR, BR)] * sc2 + sh2 + t)
        out_ref[...] = o1 + o2


def kernel(x, edge_index, W11l, W11r, b11, g11, bt11, W12l, W12r, b12, g12,
           bt12, W21l, W21r, b21, g21, bt21, W22l, W22r, b22, g22, bt22):
    src = edge_index[0]
    dst = edge_index[1]
    srcs1, dsts1 = _pad_edges(src, dst, EP1, 2)
    sd1 = jnp.stack([srcs1.reshape(NC, NS, PT1 // CHUNK, CHUNK),
                     dsts1.reshape(NC, NS, PT1 // CHUNK, CHUNK)],
                    axis=3).reshape(NC, NS, -1, K_IDX, 2, CHUNK)
    src_p, dst_p = _pad_edges(src, dst, EP2, 1)
    srcs2 = jnp.concatenate([src_p, src_p + N])
    dsts2 = jnp.concatenate([dst_p, dst_p])
    sd2 = jnp.stack([srcs2.reshape(NC, NS, PT2 // CHUNK, CHUNK),
                     dsts2.reshape(NC, NS, PT2 // CHUNK, CHUNK)],
                    axis=3).reshape(NC, NS, -1, K_IDX, 2, CHUNK)
    zeros = jnp.zeros((N_PAD, C), jnp.float32)
    zeros1 = jnp.zeros((N_PAD,), jnp.float32)

    mat = pl.BlockSpec((C, C), lambda p, i: (0, 0))
    vec = pl.BlockSpec((1, C), lambda p, i: (0, 0))
    row_p0 = pl.BlockSpec((BR, C), lambda p, i: (i * (1 - p), 0))
    row_p1 = pl.BlockSpec((BR, C), lambda p, i: (i * p, 0))
    stk_p0 = pl.BlockSpec((2, BR, C), lambda p, i: (0, i * (1 - p), 0))
    stk_p1 = pl.BlockSpec((2, BR, C), lambda p, i: (0, i * p, 0))

    def core(c, shape):
        return pl.BlockSpec(shape, lambda p, i, c=c: (c, i * (1 - p), 0))

    scratch = [pltpu.VMEM((2, N, C), jnp.float32),
               pltpu.VMEM((8, C), jnp.float32)]

    mat1 = pl.BlockSpec((C, C), lambda i: (0, 0))
    vec1 = pl.BlockSpec((1, C), lambda i: (0, 0))
    row1 = pl.BlockSpec((BR, C), lambda i: (i, 0))
    stk1 = pl.BlockSpec((2, BR, C), lambda i: (0, i, 0))

    aggx, degp = _sc_agg1(x, sd1, zeros, zeros1)
    degp3 = degp[..., None]
    rhs1 = pl.pallas_call(
        _tc_r1,
        grid=(NB,),
        in_specs=[row1, mat1, mat1, mat1, vec1, vec1],
        out_specs=stk1,
        out_shape=jax.ShapeDtypeStruct((2, N, C), jnp.float32),
        name="tc_r1",
    )(x, _MA, W11r.T, W21r.T, b11[None, :], b21[None, :])

    h = pl.pallas_call(
        _tc_a,
        grid=(2, NB),
        in_specs=[core(0, (1, BR, C)), core(1, (1, BR, C)),
                  core(0, (1, BR, 1)), core(1, (1, BR, 1)), stk_p0, mat,
                  mat, mat, vec, vec, vec, vec],
        out_specs=stk_p1,
        out_shape=jax.ShapeDtypeStruct((2, N, C), jnp.float32),
        scratch_shapes=scratch,
        name="tc_a",
    )(aggx, aggx, degp3, degp3, rhs1, _MA,
      W11l.T, W21l.T,
      g11[None, :], bt11[None, :], g21[None, :], bt21[None, :])

    (agg2,) = _sc_agg2(h.reshape(2 * N, C), sd2, zeros)
    rhs2 = pl.pallas_call(
        _tc_r2,
        grid=(NB,),
        in_specs=[stk1, mat1, mat1, vec1, vec1],
        out_specs=stk1,
        out_shape=jax.ShapeDtypeStruct((2, N, C), jnp.float32),
        name="tc_r2",
    )(h, W12r.T, W22r.T, b12[None, :], b22[None, :])

    return pl.pallas_call(
        _tc_b,
        grid=(2, NB),
        in_specs=[core(0, (1, BR, C)), core(1, (1, BR, C)),
                  core(0, (1, BR, 1)), core(1, (1, BR, 1)), stk_p0,
                  row_p1, mat, mat, mat, vec, vec, vec, vec],
        out_specs=row_p1,
        out_shape=jax.ShapeDtypeStruct((N, C), jnp.float32),
        scratch_shapes=scratch,
        name="tc_b",
    )(agg2, agg2, degp3, degp3, rhs2, x, _MA,
      W12l.T, W22l.T,
      g12[None, :], bt12[None, :], g22[None, :], bt22[None, :])
